# baseline (speedup 1.0000x reference)
import functools

import jax
import jax.numpy as jnp
from jax import lax
from jax.experimental import pallas as pl
from jax.experimental.pallas import tpu as pltpu
from jax.experimental.pallas import tpu_sc as plsc

_BIGRAM_VOCAB = 1000000
_MULT = 1009
_DIM = 32
_NW = 32
_CHUNK = 800
_NBUF = 4


def _hash_body(ids_ref, out_ref):
    ids = ids_ref[...]
    prev = jnp.concatenate(
        [jnp.zeros((1, ids.shape[1]), jnp.int32), ids[:-1, :]], axis=0
    )
    w = (prev * _MULT + ids) % _BIGRAM_VOCAB
    out_ref[...] = w - w % 512 + 4 * (w % 128) + (w % 512) // 128


def _bigram_ids(input_ids):
    batch, seq = input_ids.shape
    ids_t = jnp.transpose(input_ids, (1, 0))
    h_t = pl.pallas_call(
        _hash_body,
        out_shape=jax.ShapeDtypeStruct((seq, batch), jnp.int32),
    )(ids_t)
    return (
        h_t.reshape(seq // 4, 4, batch // 128, 128)
        .transpose(2, 0, 3, 1)
        .reshape(batch * seq)
    )


def _sc_gather(table, idx_flat):
    n = idx_flat.shape[0]
    per_w = n // _NW
    group = _NBUF * _CHUNK
    ngroups = per_w // group
    assert per_w % group == 0
    mesh = plsc.VectorSubcoreMesh(core_axis_name="c", subcore_axis_name="s")

    @functools.partial(
        pl.kernel,
        out_type=jax.ShapeDtypeStruct((n, _DIM), jnp.float32),
        mesh=mesh,
        scratch_types=[
            pltpu.VMEM((_NBUF, _CHUNK), jnp.int32),
            pltpu.VMEM((_NBUF, _CHUNK, _DIM), jnp.float32),
            pltpu.SemaphoreType.DMA((_NBUF,)),
            pltpu.SemaphoreType.DMA((_NBUF,)),
            pltpu.SemaphoreType.DMA((_NBUF,)),
        ],
        compiler_params=pltpu.CompilerParams(use_tc_tiling_on_sc=False),
    )
    def k(table_hbm, i_hbm, o_hbm, idx_v, rows_v, sem_i, sem_g, sem_o):
        wid = lax.axis_index("c") * 16 + lax.axis_index("s")
        w_base = wid * per_w

        @pl.loop(0, ngroups)
        def _(g):
            base = w_base + g * group
            idx_cps = []
            for b in range(_NBUF):
                idx_cps.append(
                    pltpu.async_copy(
                        i_hbm.at[pl.ds(base + b * _CHUNK, _CHUNK)],
                        idx_v.at[b],
                        sem_i.at[b],
                    )
                )
            gather_cps = []
            for b in range(_NBUF):
                idx_cps[b].wait()

                @pl.when(g > 0)
                def _wait_prev_out(b=b):
                    pltpu.make_async_copy(
                        rows_v.at[b],
                        o_hbm.at[pl.ds(w_base, _CHUNK)],
                        sem_o.at[b],
                    ).wait()

                gather_cps.append(
                    pltpu.async_copy(
                        table_hbm.at[idx_v.at[b]],
                        rows_v.at[b],
                        sem_g.at[b],
                    )
                )
            for b in range(_NBUF):
                gather_cps[b].wait()
                pltpu.async_copy(
                    rows_v.at[b],
                    o_hbm.at[pl.ds(base + b * _CHUNK, _CHUNK)],
                    sem_o.at[b],
                )

        for b in range(_NBUF):
            pltpu.make_async_copy(
                rows_v.at[b],
                o_hbm.at[pl.ds(w_base, _CHUNK)],
                sem_o.at[b],
            ).wait()

    return k(table, idx_flat)


_VB = 51200


def _ttr_body(in_ref, out_ref):
    x = in_ref[...]
    for m in range(_VB // 512):
        tile = jnp.concatenate(
            [x[:, m * 512 + u * 128 : m * 512 + (u + 1) * 128] for u in range(4)],
            axis=0,
        )
        out_ref[pl.ds(m * 128, 128), :] = tile.T


def _table_to_linear(emb_weight):
    vocab = emb_weight.shape[0]
    groups = (vocab + 511) // 512
    slots = ((groups * 128 + 127) // 128) * 512
    tt = jnp.transpose(emb_weight, (1, 0))
    steps = (vocab + _VB - 1) // _VB
    out = pl.pallas_call(
        _ttr_body,
        grid=(steps,),
        in_specs=[pl.BlockSpec((_DIM, _VB), lambda i: (0, i))],
        out_specs=pl.BlockSpec((_VB // 4, 128), lambda i: (i, 0)),
        out_shape=jax.ShapeDtypeStruct((slots // 4, 128), jnp.float32),
    )(tt)
    return out.reshape(slots, _DIM)


_BBLK = 128


def _transpose_body(in_ref, out_ref):
    nrr = in_ref.shape[0] // 128
    for rr in range(nrr):
        out_ref[pl.ds(rr * 128, 128), :] = in_ref[pl.ds(rr * 128, 128), :].T


def _to_final_layout(rows, batch, seq):
    n = rows.shape[0]
    xr = rows.reshape(n // 4, 128)
    rows_per_blk = _BBLK * seq // 4
    t = pl.pallas_call(
        _transpose_body,
        grid=(batch // _BBLK,),
        in_specs=[
            pl.BlockSpec((rows_per_blk, 128), lambda i: (i, 0)),
        ],
        out_specs=pl.BlockSpec((seq * _DIM, _BBLK), lambda i: (0, i)),
        out_shape=jax.ShapeDtypeStruct((seq * _DIM, batch), jnp.float32),
    )(xr)
    return jnp.transpose(t.reshape(seq, _DIM, batch), (2, 0, 1))


def kernel(input_ids, emb_weight):
    batch, seq = input_ids.shape
    ids = _bigram_ids(input_ids)
    table_lin = _table_to_linear(emb_weight)
    rows = _sc_gather(table_lin, ids.reshape(batch * seq))
    return _to_final_layout(rows, batch, seq)


# speedup vs baseline: 1.3527x; 1.3527x over previous
import functools

import jax
import jax.numpy as jnp
from jax import lax
from jax.experimental import pallas as pl
from jax.experimental.pallas import tpu as pltpu
from jax.experimental.pallas import tpu_sc as plsc

_BIGRAM_VOCAB = 1000000
_MULT = 1009
_DIM = 32
_NW = 32
_CHUNK = 800
_NBUF = 4


def _hash_body(ids_ref, out_ref):
    ids = ids_ref[...]
    prev = jnp.concatenate(
        [jnp.zeros((ids.shape[0], 1), jnp.int32), ids[:, :-1]], axis=1
    )
    w = (prev * _MULT + ids) % _BIGRAM_VOCAB
    out_ref[...] = w - w % 512 + 4 * (w % 128) + (w % 512) // 128


def _bigram_ids(input_ids):
    return pl.pallas_call(
        _hash_body,
        out_shape=jax.ShapeDtypeStruct(input_ids.shape, jnp.int32),
    )(input_ids)


def _sc_gather(table, idx_flat):
    n = idx_flat.shape[0]
    per_w = n // _NW
    group = _NBUF * _CHUNK
    ngroups = per_w // group
    assert per_w % group == 0
    mesh = plsc.VectorSubcoreMesh(core_axis_name="c", subcore_axis_name="s")

    @functools.partial(
        pl.kernel,
        out_type=jax.ShapeDtypeStruct((n, _DIM), jnp.float32),
        mesh=mesh,
        scratch_types=[
            pltpu.VMEM((_NBUF, _CHUNK), jnp.int32),
            pltpu.VMEM((_NBUF, _CHUNK, _DIM), jnp.float32),
            pltpu.SemaphoreType.DMA((_NBUF,)),
            pltpu.SemaphoreType.DMA((_NBUF,)),
            pltpu.SemaphoreType.DMA((_NBUF,)),
        ],
        compiler_params=pltpu.CompilerParams(use_tc_tiling_on_sc=False),
    )
    def k(table_hbm, i_hbm, o_hbm, idx_v, rows_v, sem_i, sem_g, sem_o):
        wid = lax.axis_index("c") * 16 + lax.axis_index("s")
        w_base = wid * per_w

        @pl.loop(0, ngroups)
        def _(g):
            base = w_base + g * group
            idx_cps = []
            for b in range(_NBUF):
                idx_cps.append(
                    pltpu.async_copy(
                        i_hbm.at[pl.ds(base + b * _CHUNK, _CHUNK)],
                        idx_v.at[b],
                        sem_i.at[b],
                    )
                )
            gather_cps = []
            for b in range(_NBUF):
                idx_cps[b].wait()

                @pl.when(g > 0)
                def _wait_prev_out(b=b):
                    pltpu.make_async_copy(
                        rows_v.at[b],
                        o_hbm.at[pl.ds(w_base, _CHUNK)],
                        sem_o.at[b],
                    ).wait()

                gather_cps.append(
                    pltpu.async_copy(
                        table_hbm.at[idx_v.at[b]],
                        rows_v.at[b],
                        sem_g.at[b],
                    )
                )
            for b in range(_NBUF):
                gather_cps[b].wait()
                pltpu.async_copy(
                    rows_v.at[b],
                    o_hbm.at[pl.ds(base + b * _CHUNK, _CHUNK)],
                    sem_o.at[b],
                )

        for b in range(_NBUF):
            pltpu.make_async_copy(
                rows_v.at[b],
                o_hbm.at[pl.ds(w_base, _CHUNK)],
                sem_o.at[b],
            ).wait()

    return k(table, idx_flat)


_VB = 51200


def _ttr_body(in_ref, out_ref):
    x = in_ref[...]
    for m in range(_VB // 512):
        tile = jnp.concatenate(
            [x[:, m * 512 + u * 128 : m * 512 + (u + 1) * 128] for u in range(4)],
            axis=0,
        )
        out_ref[pl.ds(m * 128, 128), :] = tile.T


def _table_to_linear(emb_weight):
    vocab = emb_weight.shape[0]
    groups = (vocab + 511) // 512
    slots = ((groups * 128 + 127) // 128) * 512
    tt = jnp.transpose(emb_weight, (1, 0))
    steps = (vocab + _VB - 1) // _VB
    out = pl.pallas_call(
        _ttr_body,
        grid=(steps,),
        in_specs=[pl.BlockSpec((_DIM, _VB), lambda i: (0, i))],
        out_specs=pl.BlockSpec((_VB // 4, 128), lambda i: (i, 0)),
        out_shape=jax.ShapeDtypeStruct((slots // 4, 128), jnp.float32),
    )(tt)
    return out.reshape(slots, _DIM)


_BBLK = 128


def _transpose_body(in_ref, out_ref):
    x = in_ref[...]
    seq_per = in_ref.shape[0] // _BBLK
    x3 = x.reshape(_BBLK, seq_per, 128)
    for rr in range(seq_per):
        out_ref[pl.ds(rr * 128, 128), :] = x3[:, rr, :].T


def _to_final_layout(rows, batch, seq):
    n = rows.shape[0]
    xr = rows.reshape(n // 4, 128)
    rows_per_blk = _BBLK * seq // 4
    t = pl.pallas_call(
        _transpose_body,
        grid=(batch // _BBLK,),
        in_specs=[
            pl.BlockSpec((rows_per_blk, 128), lambda i: (i, 0)),
        ],
        out_specs=pl.BlockSpec((seq * _DIM, _BBLK), lambda i: (0, i)),
        out_shape=jax.ShapeDtypeStruct((seq * _DIM, batch), jnp.float32),
    )(xr)
    return jnp.transpose(t.reshape(seq, _DIM, batch), (2, 0, 1))


def kernel(input_ids, emb_weight):
    batch, seq = input_ids.shape
    ids = _bigram_ids(input_ids)
    table_lin = _table_to_linear(emb_weight)
    rows = _sc_gather(table_lin, ids.reshape(batch * seq))
    return _to_final_layout(rows, batch, seq)


# speedup vs baseline: 1.3598x; 1.0053x over previous
import functools

import jax
import jax.numpy as jnp
from jax import lax
from jax.experimental import pallas as pl
from jax.experimental.pallas import tpu as pltpu
from jax.experimental.pallas import tpu_sc as plsc

_BIGRAM_VOCAB = 1000000
_MULT = 1009
_DIM = 32
_NW = 32
_CHUNK = 1600
_NBUF = 2


def _hash_body(ids_ref, out_ref):
    ids = ids_ref[...]
    prev = jnp.concatenate(
        [jnp.zeros((ids.shape[0], 1), jnp.int32), ids[:, :-1]], axis=1
    )
    w = (prev * _MULT + ids) % _BIGRAM_VOCAB
    out_ref[...] = w - w % 512 + 4 * (w % 128) + (w % 512) // 128


def _bigram_ids(input_ids):
    return pl.pallas_call(
        _hash_body,
        out_shape=jax.ShapeDtypeStruct(input_ids.shape, jnp.int32),
    )(input_ids)


def _sc_gather(table, idx_flat):
    n = idx_flat.shape[0]
    per_w = n // _NW
    group = _NBUF * _CHUNK
    ngroups = per_w // group
    assert per_w % group == 0
    mesh = plsc.VectorSubcoreMesh(core_axis_name="c", subcore_axis_name="s")

    @functools.partial(
        pl.kernel,
        out_type=jax.ShapeDtypeStruct((n, _DIM), jnp.float32),
        mesh=mesh,
        scratch_types=[
            pltpu.VMEM((_NBUF, _CHUNK), jnp.int32),
            pltpu.VMEM((_NBUF, _CHUNK, _DIM), jnp.float32),
            pltpu.SemaphoreType.DMA((_NBUF,)),
            pltpu.SemaphoreType.DMA((_NBUF,)),
            pltpu.SemaphoreType.DMA((_NBUF,)),
        ],
        compiler_params=pltpu.CompilerParams(use_tc_tiling_on_sc=False),
    )
    def k(table_hbm, i_hbm, o_hbm, idx_v, rows_v, sem_i, sem_g, sem_o):
        wid = lax.axis_index("c") * 16 + lax.axis_index("s")
        w_base = wid * per_w

        @pl.loop(0, ngroups)
        def _(g):
            base = w_base + g * group
            idx_cps = []
            for b in range(_NBUF):
                idx_cps.append(
                    pltpu.async_copy(
                        i_hbm.at[pl.ds(base + b * _CHUNK, _CHUNK)],
                        idx_v.at[b],
                        sem_i.at[b],
                    )
                )
            gather_cps = []
            for b in range(_NBUF):
                idx_cps[b].wait()

                @pl.when(g > 0)
                def _wait_prev_out(b=b):
                    pltpu.make_async_copy(
                        rows_v.at[b],
                        o_hbm.at[pl.ds(w_base, _CHUNK)],
                        sem_o.at[b],
                    ).wait()

                gather_cps.append(
                    pltpu.async_copy(
                        table_hbm.at[idx_v.at[b]],
                        rows_v.at[b],
                        sem_g.at[b],
                    )
                )
            for b in range(_NBUF):
                gather_cps[b].wait()
                pltpu.async_copy(
                    rows_v.at[b],
                    o_hbm.at[pl.ds(base + b * _CHUNK, _CHUNK)],
                    sem_o.at[b],
                )

        for b in range(_NBUF):
            pltpu.make_async_copy(
                rows_v.at[b],
                o_hbm.at[pl.ds(w_base, _CHUNK)],
                sem_o.at[b],
            ).wait()

    return k(table, idx_flat)


_VB = 51200


def _ttr_body(in_ref, out_ref):
    x = in_ref[...]
    for m in range(_VB // 512):
        tile = jnp.concatenate(
            [x[:, m * 512 + u * 128 : m * 512 + (u + 1) * 128] for u in range(4)],
            axis=0,
        )
        out_ref[pl.ds(m * 128, 128), :] = tile.T


def _table_to_linear(emb_weight):
    vocab = emb_weight.shape[0]
    groups = (vocab + 511) // 512
    slots = ((groups * 128 + 127) // 128) * 512
    tt = jnp.transpose(emb_weight, (1, 0))
    steps = (vocab + _VB - 1) // _VB
    out = pl.pallas_call(
        _ttr_body,
        grid=(steps,),
        in_specs=[pl.BlockSpec((_DIM, _VB), lambda i: (0, i))],
        out_specs=pl.BlockSpec((_VB // 4, 128), lambda i: (i, 0)),
        out_shape=jax.ShapeDtypeStruct((slots // 4, 128), jnp.float32),
    )(tt)
    return out.reshape(slots, _DIM)


_BBLK = 128


def _transpose_body(in_ref, out_ref):
    x = in_ref[...]
    seq_per = in_ref.shape[0] // _BBLK
    x3 = x.reshape(_BBLK, seq_per, 128)
    for rr in range(seq_per):
        out_ref[pl.ds(rr * 128, 128), :] = x3[:, rr, :].T


def _to_final_layout(rows, batch, seq):
    n = rows.shape[0]
    xr = rows.reshape(n // 4, 128)
    rows_per_blk = _BBLK * seq // 4
    t = pl.pallas_call(
        _transpose_body,
        grid=(batch // _BBLK,),
        in_specs=[
            pl.BlockSpec((rows_per_blk, 128), lambda i: (i, 0)),
        ],
        out_specs=pl.BlockSpec((seq * _DIM, _BBLK), lambda i: (0, i)),
        out_shape=jax.ShapeDtypeStruct((seq * _DIM, batch), jnp.float32),
    )(xr)
    return jnp.transpose(t.reshape(seq, _DIM, batch), (2, 0, 1))


def kernel(input_ids, emb_weight):
    batch, seq = input_ids.shape
    ids = _bigram_ids(input_ids)
    table_lin = _table_to_linear(emb_weight)
    rows = _sc_gather(table_lin, ids.reshape(batch * seq))
    return _to_final_layout(rows, batch, seq)
